# baseline (speedup 1.0000x reference)
import functools

import jax
import jax.numpy as jnp
from jax.experimental import pallas as pl
from jax.experimental.pallas import tpu as pltpu

_N_OUT = 4096


def _mm_body(x_ref, w_hbm, b_ref, o_ref, wb_ref, t0, t1, s0, s1, *,
             ck, n_chunks):
    i = pl.program_id(0)

    @pl.when(i == 0)
    def _():
        temps = (t0, t1)
        sems = (s0, s1)

        def copy(c, slot):
            return pltpu.make_async_copy(
                w_hbm.at[:, pl.ds(c * ck, ck)], temps[slot], sems[slot])

        copy(0, 0).start()
        for c in range(n_chunks):
            slot = c % 2
            if c + 1 < n_chunks:
                copy(c + 1, 1 - slot).start()
            copy(c, slot).wait()
            wb_ref[:, c * ck:(c + 1) * ck] = temps[slot][...].astype(
                jnp.bfloat16)

    xb = x_ref[...].astype(jnp.bfloat16)
    o_ref[...] = (
        jnp.dot(xb, wb_ref[...], preferred_element_type=jnp.float32)
        + b_ref[...]
    )


def _pick_tile(total, cap, align):
    best = align
    t = align
    while t <= min(total, cap):
        if total % t == 0:
            best = t
        t += align
    return best


def kernel(x, w_t, b2):
    B, K = x.shape
    Kp, Np = w_t.shape
    assert Kp == K

    bm = _pick_tile(B, 128, 8)
    ck = _pick_tile(Np, 256, 128)
    n_chunks = Np // ck
    grid = (B // bm,)

    out = pl.pallas_call(
        functools.partial(_mm_body, ck=ck, n_chunks=n_chunks),
        grid=grid,
        in_specs=[
            pl.BlockSpec((bm, K), lambda i: (i, 0)),
            pl.BlockSpec(memory_space=pl.ANY),
            pl.BlockSpec((1, Np), lambda i: (0, 0)),
        ],
        out_specs=pl.BlockSpec((bm, Np), lambda i: (i, 0)),
        out_shape=jax.ShapeDtypeStruct((B, Np), jnp.float32),
        scratch_shapes=[
            pltpu.VMEM((K, Np), jnp.bfloat16),
            pltpu.VMEM((K, ck), jnp.float32),
            pltpu.VMEM((K, ck), jnp.float32),
            pltpu.SemaphoreType.DMA,
            pltpu.SemaphoreType.DMA,
        ],
        compiler_params=pltpu.CompilerParams(
            dimension_semantics=("arbitrary",),
            vmem_limit_bytes=60000 * 1024,
        ),
        cost_estimate=pl.CostEstimate(
            flops=2 * B * Np * K,
            transcendentals=0,
            bytes_accessed=4 * B * K + 4 * K * Np + 4 * B * Np,
        ),
    )(x, w_t, b2)

    if Np != _N_OUT:
        out = out[:, :_N_OUT]
    return out


# speedup vs baseline: 1.0438x; 1.0438x over previous
import functools

import jax
import jax.numpy as jnp
from jax.experimental import pallas as pl
from jax.experimental.pallas import tpu as pltpu

_N_OUT = 4096


def _mm_body(x_ref, w_hbm, b_ref, o_ref, wb_ref, t0, t1, s0, s1, *,
             ck, n_chunks):
    i = pl.program_id(0)

    @pl.when(i == 0)
    def _():
        temps = (t0, t1)
        sems = (s0, s1)

        def copy(c, slot):
            return pltpu.make_async_copy(
                w_hbm.at[:, pl.ds(c * ck, ck)], temps[slot], sems[slot])

        copy(0, 0).start()
        xb0 = x_ref[...].astype(jnp.bfloat16)
        for c in range(n_chunks):
            slot = c % 2
            if c + 1 < n_chunks:
                copy(c + 1, 1 - slot).start()
            copy(c, slot).wait()
            wc = temps[slot][...].astype(jnp.bfloat16)
            wb_ref[:, c * ck:(c + 1) * ck] = wc
            o_ref[:, c * ck:(c + 1) * ck] = (
                jnp.dot(xb0, wc, preferred_element_type=jnp.float32)
                + b_ref[:, c * ck:(c + 1) * ck]
            )

    @pl.when(i != 0)
    def _():
        xb = x_ref[...].astype(jnp.bfloat16)
        o_ref[...] = (
            jnp.dot(xb, wb_ref[...], preferred_element_type=jnp.float32)
            + b_ref[...]
        )


def _pick_tile(total, cap, align):
    best = align
    t = align
    while t <= min(total, cap):
        if total % t == 0:
            best = t
        t += align
    return best


def kernel(x, w_t, b2):
    B, K = x.shape
    Kp, Np = w_t.shape
    assert Kp == K

    bm = _pick_tile(B, 256, 8)
    ck = _pick_tile(Np, 256, 128)
    n_chunks = Np // ck
    grid = (B // bm,)

    out = pl.pallas_call(
        functools.partial(_mm_body, ck=ck, n_chunks=n_chunks),
        grid=grid,
        in_specs=[
            pl.BlockSpec((bm, K), lambda i: (i, 0)),
            pl.BlockSpec(memory_space=pl.ANY),
            pl.BlockSpec((1, Np), lambda i: (0, 0)),
        ],
        out_specs=pl.BlockSpec((bm, Np), lambda i: (i, 0)),
        out_shape=jax.ShapeDtypeStruct((B, Np), jnp.float32),
        scratch_shapes=[
            pltpu.VMEM((K, Np), jnp.bfloat16),
            pltpu.VMEM((K, ck), jnp.float32),
            pltpu.VMEM((K, ck), jnp.float32),
            pltpu.SemaphoreType.DMA,
            pltpu.SemaphoreType.DMA,
        ],
        compiler_params=pltpu.CompilerParams(
            dimension_semantics=("arbitrary",),
            vmem_limit_bytes=60000 * 1024,
        ),
        cost_estimate=pl.CostEstimate(
            flops=2 * B * Np * K,
            transcendentals=0,
            bytes_accessed=4 * B * K + 4 * K * Np + 4 * B * Np,
        ),
    )(x, w_t, b2)

    if Np != _N_OUT:
        out = out[:, :_N_OUT]
    return out
